# Initial kernel scaffold; baseline (speedup 1.0000x reference)
import jax
import jax.numpy as jnp
from jax.experimental import pallas as pl


def kernel(embedding, idx_0, idx_1, idx_2, idx_3, W_0, b_0, W_1, b_1, W_2, b_2, W_3, b_3):
    raise NotImplementedError("write your pallas kernel here")


# speedup vs baseline: 292.9186x; 292.9186x over previous
import functools

import jax
import jax.numpy as jnp
from jax import lax
from jax.experimental import pallas as pl
from jax.experimental.pallas import tpu as pltpu
from jax.experimental.pallas import tpu_sc as plsc

NUM_NODES = 100000
D_IN = 128
NG = 8192
DIMS = (64, 128, 32, 256)
MAX_D = 256
SUB = 32
NSUB = MAX_D // SUB
OUT_ROWS = NUM_NODES * NSUB

NC = 2
NS = 16
LANES = 16

B_ALL = 4 * NG
G_WORKERS = NC * NS
G_PER_W = B_ALL // G_WORKERS
G_CHUNK = 128
G_NCH = G_PER_W // G_CHUNK

_gather_mesh = plsc.VectorSubcoreMesh(core_axis_name="c", subcore_axis_name="s")


@functools.partial(
    pl.kernel,
    out_type=jax.ShapeDtypeStruct((B_ALL, D_IN), jnp.float32),
    mesh=_gather_mesh,
    scratch_types=[
        pltpu.VMEM((G_CHUNK,), jnp.int32),
        pltpu.VMEM((G_CHUNK, D_IN), jnp.float32),
        pltpu.SemaphoreType.DMA,
    ],
)
def _sc_gather(emb, idx, out, idxv, rows, sem):
    wid = lax.axis_index("s") * NC + lax.axis_index("c")
    base = wid * G_PER_W
    for ch in range(G_NCH):
        off = base + ch * G_CHUNK
        pltpu.sync_copy(idx.at[pl.ds(off, G_CHUNK)], idxv)
        pltpu.async_copy(emb.at[idxv], rows, sem).wait()
        pltpu.sync_copy(rows, out.at[pl.ds(off, G_CHUNK)])


MM_BLK = 1024


def _mm_body(g_ref, w_ref, b_ref, y_ref):
    y_ref[...] = (
        jnp.dot(g_ref[0], w_ref[0], preferred_element_type=jnp.float32)
        + b_ref[0]
    )[None]


_tc_matmul = pl.pallas_call(
    _mm_body,
    grid=(4, NG // MM_BLK),
    in_specs=[
        pl.BlockSpec((1, MM_BLK, D_IN), lambda t, i: (t, i, 0)),
        pl.BlockSpec((1, D_IN, MAX_D), lambda t, i: (t, 0, 0)),
        pl.BlockSpec((1, 1, MAX_D), lambda t, i: (t, 0, 0)),
    ],
    out_specs=pl.BlockSpec((1, MM_BLK, MAX_D), lambda t, i: (t, i, 0)),
    out_shape=jax.ShapeDtypeStruct((4, NG, MAX_D), jnp.float32),
)


S_WORKERS = NS
S_PER_W = NG // S_WORKERS
S_CHUNK = 128
S_NCH = S_PER_W // S_CHUNK
Z_PER_W = OUT_ROWS // S_WORKERS
Z_CHUNK = 2000
Z_NCH = Z_PER_W // Z_CHUNK

_scatter_mesh = plsc.VectorSubcoreMesh(
    core_axis_name="c", subcore_axis_name="s", num_cores=1
)


@functools.partial(
    pl.kernel,
    out_type=jax.ShapeDtypeStruct((OUT_ROWS, SUB), jnp.float32),
    mesh=_scatter_mesh,
    scratch_types=[
        pltpu.VMEM((Z_CHUNK, SUB), jnp.float32),
        pltpu.VMEM((S_PER_W,), jnp.int32),
        pltpu.VMEM((NSUB, S_CHUNK), jnp.int32),
        pltpu.VMEM((NSUB, S_CHUNK, SUB), jnp.float32),
        pltpu.SemaphoreType.DMA,
    ],
    compiler_params=pltpu.CompilerParams(use_tc_tiling_on_sc=False),
)
def _sc_scatter(i0, i1, i2, i3, y_r, zsrc, out, zbuf, idxv, idx_sc, srcbuf,
                sem):
    wid = lax.axis_index("s")
    pltpu.sync_copy(zsrc, zbuf)
    zbase = wid * Z_PER_W
    handles = []
    for zc in range(Z_NCH):
        handles.append(
            pltpu.async_copy(zbuf, out.at[pl.ds(zbase + zc * Z_CHUNK, Z_CHUNK)],
                             sem))
    for h in handles:
        h.wait()
    plsc.subcore_barrier()

    idx_refs = (i0, i1, i2, i3)
    for t in range(4):
        nsub_t = DIMS[t] // SUB
        row0 = wid * S_PER_W
        pltpu.sync_copy(idx_refs[t].at[pl.ds(row0, S_PER_W)], idxv)
        for c in range(S_NCH):
            for k in range(S_CHUNK // LANES):
                v8 = idxv[pl.ds(c * S_CHUNK + k * LANES, LANES)] * NSUB
                for j in range(nsub_t):
                    idx_sc[j, pl.ds(k * LANES, LANES)] = v8 + j
            for j in range(nsub_t):
                pltpu.sync_copy(
                    y_r.at[t, pl.ds(row0 + c * S_CHUNK, S_CHUNK), j],
                    srcbuf.at[j])
            handles = []
            for j in range(nsub_t):
                handles.append(
                    pltpu.async_copy(srcbuf.at[j], out.at[idx_sc.at[j]], sem))
            for h in handles:
                h.wait()
        if t != 3:
            plsc.subcore_barrier()


def kernel(embedding, idx_0, idx_1, idx_2, idx_3,
           W_0, b_0, W_1, b_1, W_2, b_2, W_3, b_3):
    idx_all = jnp.concatenate([idx_0, idx_1, idx_2, idx_3])
    g = _sc_gather(embedding, idx_all)
    g4 = g.reshape(4, NG, D_IN)

    Ws = (W_0, W_1, W_2, W_3)
    bs = (b_0, b_1, b_2, b_3)
    W_pad = jnp.zeros((4, D_IN, MAX_D), jnp.float32)
    b_pad = jnp.zeros((4, 1, MAX_D), jnp.float32)
    for t in range(4):
        W_pad = W_pad.at[t, :, :DIMS[t]].set(Ws[t])
        b_pad = b_pad.at[t, 0, :DIMS[t]].set(bs[t])

    y4 = _tc_matmul(g4, W_pad, b_pad)
    y_r = y4.reshape(4, NG, NSUB, SUB)

    zsrc = jnp.zeros((Z_CHUNK, SUB), jnp.float32)
    x_flat = _sc_scatter(idx_0, idx_1, idx_2, idx_3, y_r, zsrc)
    return x_flat.reshape(NUM_NODES, MAX_D)


# speedup vs baseline: 635.8801x; 2.1708x over previous
import functools

import jax
import jax.numpy as jnp
from jax import lax
from jax.experimental import pallas as pl
from jax.experimental.pallas import tpu as pltpu
from jax.experimental.pallas import tpu_sc as plsc

NUM_NODES = 100000
D_IN = 128
NG = 8192
DIMS = (64, 128, 32, 256)
MAX_D = 256

NC = 2
NS = 16
LANES = 16

B_ALL = 4 * NG
G_WORKERS = NC * NS
G_PER_W = B_ALL // G_WORKERS
G_CHUNK = 128
G_NCH = G_PER_W // G_CHUNK

_gather_mesh = plsc.VectorSubcoreMesh(core_axis_name="c", subcore_axis_name="s")


@functools.partial(
    pl.kernel,
    out_type=jax.ShapeDtypeStruct((B_ALL, D_IN), jnp.float32),
    mesh=_gather_mesh,
    scratch_types=[
        pltpu.VMEM((G_CHUNK,), jnp.int32),
        pltpu.VMEM((G_CHUNK, D_IN), jnp.float32),
        pltpu.SemaphoreType.DMA,
    ],
    compiler_params=pltpu.CompilerParams(use_tc_tiling_on_sc=True),
)
def _sc_gather(emb, idx, out, idxv, rows, sem):
    wid = lax.axis_index("s") * NC + lax.axis_index("c")
    base = wid * G_PER_W
    for ch in range(G_NCH):
        off = base + ch * G_CHUNK
        pltpu.sync_copy(idx.at[pl.ds(off, G_CHUNK)], idxv)
        pltpu.async_copy(emb.at[idxv], rows, sem).wait()
        pltpu.sync_copy(rows, out.at[pl.ds(off, G_CHUNK)])


MM_BLK = 1024


def _mm_body(g_ref, w_ref, b_ref, y_ref):
    y_ref[...] = (
        jnp.dot(g_ref[0], w_ref[0], preferred_element_type=jnp.float32)
        + b_ref[0]
    )[None]


_tc_matmul = pl.pallas_call(
    _mm_body,
    grid=(4, NG // MM_BLK),
    in_specs=[
        pl.BlockSpec((1, MM_BLK, D_IN), lambda t, i: (t, i, 0)),
        pl.BlockSpec((1, D_IN, MAX_D), lambda t, i: (t, 0, 0)),
        pl.BlockSpec((1, 1, MAX_D), lambda t, i: (t, 0, 0)),
    ],
    out_specs=pl.BlockSpec((1, MM_BLK, MAX_D), lambda t, i: (t, i, 0)),
    out_shape=jax.ShapeDtypeStruct((4, NG, MAX_D), jnp.float32),
)


S_WORKERS = NS
S_PER_W = NG // S_WORKERS
S_CHUNK = 128
S_NCH = S_PER_W // S_CHUNK
Z_CHUNK = 128
Z_FULL = NUM_NODES // Z_CHUNK
Z_TAIL_ROWS = NUM_NODES - Z_FULL * Z_CHUNK
Z_EVEN = Z_FULL // S_WORKERS * S_WORKERS

_scatter_mesh = plsc.VectorSubcoreMesh(
    core_axis_name="c", subcore_axis_name="s", num_cores=1
)


@functools.partial(
    pl.kernel,
    out_type=jax.ShapeDtypeStruct((NUM_NODES, MAX_D), jnp.float32),
    mesh=_scatter_mesh,
    scratch_types=[
        pltpu.VMEM((Z_CHUNK, MAX_D), jnp.float32),
        pltpu.VMEM((S_CHUNK, MAX_D), jnp.float32),
        pltpu.VMEM((8, S_CHUNK), jnp.int32),
        pltpu.SemaphoreType.DMA,
    ],
    compiler_params=pltpu.CompilerParams(use_tc_tiling_on_sc=True),
)
def _sc_scatter(i0, i1, i2, i3, y4, zsrc, out, zbuf, gbuf, idx_sc, sem):
    wid = lax.axis_index("s")
    pltpu.sync_copy(zsrc, zbuf)
    handles = []
    for k in range(Z_EVEN // S_WORKERS):
        ch = wid + k * S_WORKERS
        handles.append(
            pltpu.async_copy(zbuf, out.at[pl.ds(ch * Z_CHUNK, Z_CHUNK)], sem))
    for h in handles:
        h.wait()
    @pl.when(wid < Z_FULL - Z_EVEN)
    def _():
        pltpu.async_copy(
            zbuf, out.at[pl.ds((Z_EVEN + wid) * Z_CHUNK, Z_CHUNK)], sem
        ).wait()

    @pl.when(wid == S_WORKERS - 1)
    def _():
        pltpu.async_copy(
            zbuf.at[pl.ds(0, Z_TAIL_ROWS)],
            out.at[pl.ds(Z_FULL * Z_CHUNK, Z_TAIL_ROWS)], sem
        ).wait()

    plsc.subcore_barrier()

    idx_refs = (i0, i1, i2, i3)
    row0 = wid * S_PER_W
    for t in range(4):
        for c in range(S_NCH):
            pltpu.sync_copy(idx_refs[t].at[pl.ds(row0 + c * S_CHUNK, S_CHUNK)],
                            idx_sc.at[c])
        for c in range(S_NCH):
            rows = pl.ds(row0 + c * S_CHUNK, S_CHUNK)
            pltpu.sync_copy(y4.at[t, rows], zbuf)
            if t == 2:
                pltpu.async_copy(out.at[idx_sc.at[c]], gbuf, sem).wait()

                def _patch(r, carry):
                    gbuf[r, pl.ds(0, LANES)] = zbuf[r, pl.ds(0, LANES)]
                    gbuf[r, pl.ds(LANES, LANES)] = zbuf[r, pl.ds(LANES, LANES)]
                    return carry

                lax.fori_loop(0, S_CHUNK, _patch, 0)
                pltpu.async_copy(gbuf, out.at[idx_sc.at[c]], sem).wait()
            else:
                pltpu.async_copy(zbuf, out.at[idx_sc.at[c]], sem).wait()
        if t != 3:
            plsc.subcore_barrier()


def kernel(embedding, idx_0, idx_1, idx_2, idx_3,
           W_0, b_0, W_1, b_1, W_2, b_2, W_3, b_3):
    idx_all = jnp.concatenate([idx_0, idx_1, idx_2, idx_3])
    g = _sc_gather(embedding, idx_all)
    g4 = g.reshape(4, NG, D_IN)

    Ws = (W_0, W_1, W_2, W_3)
    bs = (b_0, b_1, b_2, b_3)
    W_pad = jnp.zeros((4, D_IN, MAX_D), jnp.float32)
    b_pad = jnp.zeros((4, 1, MAX_D), jnp.float32)
    for t in range(4):
        W_pad = W_pad.at[t, :, :DIMS[t]].set(Ws[t])
        b_pad = b_pad.at[t, 0, :DIMS[t]].set(bs[t])

    y4 = _tc_matmul(g4, W_pad, b_pad)

    zsrc = jnp.zeros((Z_CHUNK, MAX_D), jnp.float32)
    return _sc_scatter(idx_0, idx_1, idx_2, idx_3, y4, zsrc)


# speedup vs baseline: 751.0321x; 1.1811x over previous
import functools

import jax
import jax.numpy as jnp
from jax import lax
from jax.experimental import pallas as pl
from jax.experimental.pallas import tpu as pltpu
from jax.experimental.pallas import tpu_sc as plsc

NUM_NODES = 100000
D_IN = 128
NG = 8192
DIMS = (64, 128, 32, 256)
MAX_D = 256

NC = 2
NS = 16
LANES = 16

B_ALL = 4 * NG
G_WORKERS = NC * NS
G_PER_W = B_ALL // G_WORKERS
G_CHUNK = 128
G_NCH = G_PER_W // G_CHUNK

_gather_mesh = plsc.VectorSubcoreMesh(core_axis_name="c", subcore_axis_name="s")


@functools.partial(
    pl.kernel,
    out_type=jax.ShapeDtypeStruct((B_ALL, D_IN), jnp.float32),
    mesh=_gather_mesh,
    scratch_types=[
        pltpu.VMEM((G_CHUNK,), jnp.int32),
        pltpu.VMEM((G_CHUNK, D_IN), jnp.float32),
        pltpu.SemaphoreType.DMA,
    ],
    compiler_params=pltpu.CompilerParams(use_tc_tiling_on_sc=True),
)
def _sc_gather(emb, idx, out, idxv, rows, sem):
    wid = lax.axis_index("s") * NC + lax.axis_index("c")
    base = wid * G_PER_W
    for ch in range(G_NCH):
        off = base + ch * G_CHUNK
        pltpu.sync_copy(idx.at[pl.ds(off, G_CHUNK)], idxv)
        pltpu.async_copy(emb.at[idxv], rows, sem).wait()
        pltpu.sync_copy(rows, out.at[pl.ds(off, G_CHUNK)])


MM_BLK = 1024


def _mm_body(g_ref, w_ref, b_ref, y_ref):
    y_ref[...] = (
        jnp.dot(g_ref[0], w_ref[0], preferred_element_type=jnp.float32)
        + b_ref[0]
    )[None]


_tc_matmul = pl.pallas_call(
    _mm_body,
    grid=(4, NG // MM_BLK),
    in_specs=[
        pl.BlockSpec((1, MM_BLK, D_IN), lambda t, i: (t, i, 0)),
        pl.BlockSpec((1, D_IN, MAX_D), lambda t, i: (t, 0, 0)),
        pl.BlockSpec((1, 1, MAX_D), lambda t, i: (t, 0, 0)),
    ],
    out_specs=pl.BlockSpec((1, MM_BLK, MAX_D), lambda t, i: (t, i, 0)),
    out_shape=jax.ShapeDtypeStruct((4, NG, MAX_D), jnp.float32),
)


S_WORKERS = NS
S_PER_W = NG // S_WORKERS
S_CHUNK = 128
S_NCH = S_PER_W // S_CHUNK
HALF = MAX_D // 2
Z_CHUNK = 512
Z_FULL = NUM_NODES // Z_CHUNK
Z_TAIL_ROWS = NUM_NODES - Z_FULL * Z_CHUNK
Z_EVEN = Z_FULL // S_WORKERS * S_WORKERS

_scatter_mesh = plsc.VectorSubcoreMesh(core_axis_name="c", subcore_axis_name="s")


@functools.partial(
    pl.kernel,
    out_type=jax.ShapeDtypeStruct((NUM_NODES, MAX_D), jnp.float32),
    mesh=_scatter_mesh,
    scratch_types=[
        pltpu.VMEM((Z_CHUNK, HALF), jnp.float32),
        pltpu.VMEM((S_CHUNK, HALF), jnp.float32),
        pltpu.VMEM((S_CHUNK, HALF), jnp.float32),
        pltpu.VMEM((8, S_CHUNK), jnp.int32),
        pltpu.SemaphoreType.DMA,
    ],
    compiler_params=pltpu.CompilerParams(use_tc_tiling_on_sc=True),
)
def _sc_scatter(i0, i1, i2, i3, y4, zsrc, out, zbuf, sbuf, gbuf, idx_sc, sem):
    cid = lax.axis_index("c")
    wid = lax.axis_index("s")
    row0 = wid * S_PER_W

    def _stage_idx(idx_ref):
        for c in range(S_NCH):
            pltpu.sync_copy(idx_ref.at[pl.ds(row0 + c * S_CHUNK, S_CHUNK)],
                            idx_sc.at[c])

    def _scatter_type(t, col0):
        cols = pl.ds(col0, HALF)
        _stage_idx((i0, i1, i2, i3)[t])
        for c in range(S_NCH):
            rows = pl.ds(row0 + c * S_CHUNK, S_CHUNK)
            pltpu.sync_copy(y4.at[t, rows, cols], sbuf)
            if t == 2:
                pltpu.async_copy(out.at[idx_sc.at[c], cols], gbuf, sem).wait()

                def _patch(r, carry):
                    gbuf[r, pl.ds(0, LANES)] = sbuf[r, pl.ds(0, LANES)]
                    gbuf[r, pl.ds(LANES, LANES)] = sbuf[r, pl.ds(LANES, LANES)]
                    return carry

                lax.fori_loop(0, S_CHUNK, _patch, 0)
                pltpu.async_copy(gbuf, out.at[idx_sc.at[c], cols], sem).wait()
            else:
                pltpu.async_copy(sbuf, out.at[idx_sc.at[c], cols], sem).wait()

    pltpu.sync_copy(zsrc, zbuf)

    def _zero(col0):
        cols = pl.ds(col0, HALF)
        handles = []
        for k in range(Z_EVEN // S_WORKERS):
            ch = wid + k * S_WORKERS
            handles.append(
                pltpu.async_copy(zbuf, out.at[pl.ds(ch * Z_CHUNK, Z_CHUNK),
                                              cols], sem))
        for h in handles:
            h.wait()
        @pl.when(wid < Z_FULL - Z_EVEN)
        def _():
            pltpu.async_copy(
                zbuf, out.at[pl.ds((Z_EVEN + wid) * Z_CHUNK, Z_CHUNK), cols],
                sem).wait()

        @pl.when(wid == S_WORKERS - 1)
        def _():
            pltpu.async_copy(
                zbuf.at[pl.ds(0, Z_TAIL_ROWS)],
                out.at[pl.ds(Z_FULL * Z_CHUNK, Z_TAIL_ROWS), cols], sem
            ).wait()

    @pl.when(cid == 0)
    def _():
        _zero(0)

    @pl.when(cid == 1)
    def _():
        _zero(HALF)

    plsc.subcore_barrier()

    @pl.when(cid == 0)
    def _():
        _scatter_type(0, 0)

    @pl.when(cid == 1)
    def _():
        cols = pl.ds(HALF, HALF)
        _stage_idx(i3)
        for c in range(S_NCH):
            rows = pl.ds(row0 + c * S_CHUNK, S_CHUNK)
            pltpu.sync_copy(y4.at[3, rows, cols], sbuf)
            pltpu.async_copy(sbuf, out.at[idx_sc.at[c], cols], sem).wait()

    plsc.subcore_barrier()

    @pl.when(cid == 0)
    def _():
        _scatter_type(1, 0)

    plsc.subcore_barrier()

    @pl.when(cid == 0)
    def _():
        _scatter_type(2, 0)

    plsc.subcore_barrier()

    @pl.when(cid == 0)
    def _():
        _scatter_type(3, 0)


def kernel(embedding, idx_0, idx_1, idx_2, idx_3,
           W_0, b_0, W_1, b_1, W_2, b_2, W_3, b_3):
    idx_all = jnp.concatenate([idx_0, idx_1, idx_2, idx_3])
    g = _sc_gather(embedding, idx_all)
    g4 = g.reshape(4, NG, D_IN)

    Ws = (W_0, W_1, W_2, W_3)
    bs = (b_0, b_1, b_2, b_3)
    W_pad = jnp.zeros((4, D_IN, MAX_D), jnp.float32)
    b_pad = jnp.zeros((4, 1, MAX_D), jnp.float32)
    for t in range(4):
        W_pad = W_pad.at[t, :, :DIMS[t]].set(Ws[t])
        b_pad = b_pad.at[t, 0, :DIMS[t]].set(bs[t])

    y4 = _tc_matmul(g4, W_pad, b_pad)

    zsrc = jnp.zeros((Z_CHUNK, HALF), jnp.float32)
    return _sc_scatter(idx_0, idx_1, idx_2, idx_3, y4, zsrc)


# speedup vs baseline: 792.3486x; 1.0550x over previous
import functools

import jax
import jax.numpy as jnp
from jax import lax
from jax.experimental import pallas as pl
from jax.experimental.pallas import tpu as pltpu
from jax.experimental.pallas import tpu_sc as plsc

NUM_NODES = 100000
D_IN = 128
NG = 8192
DIMS = (64, 128, 32, 256)
MAX_D = 256

NC = 2
NS = 16
LANES = 16

B_ALL = 4 * NG
G_WORKERS = NC * NS
G_PER_W = B_ALL // G_WORKERS
G_CHUNK = 128
G_NCH = G_PER_W // G_CHUNK

_gather_mesh = plsc.VectorSubcoreMesh(core_axis_name="c", subcore_axis_name="s")


@functools.partial(
    pl.kernel,
    out_type=jax.ShapeDtypeStruct((B_ALL, D_IN), jnp.float32),
    mesh=_gather_mesh,
    scratch_types=[
        pltpu.VMEM((G_CHUNK,), jnp.int32),
        pltpu.VMEM((G_CHUNK, D_IN), jnp.float32),
        pltpu.SemaphoreType.DMA,
    ],
    compiler_params=pltpu.CompilerParams(use_tc_tiling_on_sc=True),
)
def _sc_gather(emb, idx, out, idxv, rows, sem):
    wid = lax.axis_index("s") * NC + lax.axis_index("c")
    base = wid * G_PER_W
    for ch in range(G_NCH):
        off = base + ch * G_CHUNK
        pltpu.sync_copy(idx.at[pl.ds(off, G_CHUNK)], idxv)
        pltpu.async_copy(emb.at[idxv], rows, sem).wait()
        pltpu.sync_copy(rows, out.at[pl.ds(off, G_CHUNK)])


MM_BLK = 1024


def _mm_body(g_ref, w_ref, b_ref, y_ref):
    y_ref[...] = (
        jnp.dot(g_ref[0], w_ref[0], preferred_element_type=jnp.float32)
        + b_ref[0]
    )[None]


_tc_matmul = pl.pallas_call(
    _mm_body,
    grid=(4, NG // MM_BLK),
    in_specs=[
        pl.BlockSpec((1, MM_BLK, D_IN), lambda t, i: (t, i, 0)),
        pl.BlockSpec((1, D_IN, MAX_D), lambda t, i: (t, 0, 0)),
        pl.BlockSpec((1, 1, MAX_D), lambda t, i: (t, 0, 0)),
    ],
    out_specs=pl.BlockSpec((1, MM_BLK, MAX_D), lambda t, i: (t, i, 0)),
    out_shape=jax.ShapeDtypeStruct((4, NG, MAX_D), jnp.float32),
)


S_WORKERS = NS
S_PER_W = NG // S_WORKERS
S_CHUNK = 128
S_NCH = S_PER_W // S_CHUNK
HALF = MAX_D // 2

_scatter_mesh = plsc.VectorSubcoreMesh(core_axis_name="c", subcore_axis_name="s")


@functools.partial(
    pl.kernel,
    mesh=_scatter_mesh,
    scratch_types=[
        pltpu.VMEM((S_CHUNK, HALF), jnp.float32),
        pltpu.VMEM((S_CHUNK, HALF), jnp.float32),
        pltpu.VMEM((8, S_CHUNK), jnp.int32),
        pltpu.SemaphoreType.DMA,
    ],
    compiler_params=pltpu.CompilerParams(use_tc_tiling_on_sc=True),
)
def _sc_scatter(i0, i1, i2, i3, y4, out, sbuf, gbuf, idx_sc, sem):
    cid = lax.axis_index("c")
    wid = lax.axis_index("s")
    row0 = wid * S_PER_W

    def _stage_idx(idx_ref):
        for c in range(S_NCH):
            pltpu.sync_copy(idx_ref.at[pl.ds(row0 + c * S_CHUNK, S_CHUNK)],
                            idx_sc.at[c])

    def _scatter_type(t, col0):
        cols = pl.ds(col0, HALF)
        _stage_idx((i0, i1, i2, i3)[t])
        for c in range(S_NCH):
            rows = pl.ds(row0 + c * S_CHUNK, S_CHUNK)
            pltpu.sync_copy(y4.at[t, rows, cols], sbuf)
            if t == 2:
                pltpu.async_copy(out.at[idx_sc.at[c], cols], gbuf, sem).wait()

                def _patch(r, carry):
                    gbuf[r, pl.ds(0, LANES)] = sbuf[r, pl.ds(0, LANES)]
                    gbuf[r, pl.ds(LANES, LANES)] = sbuf[r, pl.ds(LANES, LANES)]
                    return carry

                lax.fori_loop(0, S_CHUNK, _patch, 0)
                pltpu.async_copy(gbuf, out.at[idx_sc.at[c], cols], sem).wait()
            else:
                pltpu.async_copy(sbuf, out.at[idx_sc.at[c], cols], sem).wait()

    @pl.when(cid == 0)
    def _():
        _scatter_type(0, 0)

    @pl.when(cid == 1)
    def _():
        cols = pl.ds(HALF, HALF)
        _stage_idx(i3)
        for c in range(S_NCH):
            rows = pl.ds(row0 + c * S_CHUNK, S_CHUNK)
            pltpu.sync_copy(y4.at[3, rows, cols], sbuf)
            pltpu.async_copy(sbuf, out.at[idx_sc.at[c], cols], sem).wait()

    plsc.subcore_barrier()

    @pl.when(cid == 0)
    def _():
        _scatter_type(1, 0)

    plsc.subcore_barrier()

    @pl.when(cid == 0)
    def _():
        _scatter_type(2, 0)

    plsc.subcore_barrier()

    @pl.when(cid == 0)
    def _():
        _scatter_type(3, 0)


def kernel(embedding, idx_0, idx_1, idx_2, idx_3,
           W_0, b_0, W_1, b_1, W_2, b_2, W_3, b_3):
    idx_all = jnp.concatenate([idx_0, idx_1, idx_2, idx_3])
    g = _sc_gather(embedding, idx_all)
    g4 = g.reshape(4, NG, D_IN)

    Ws = (W_0, W_1, W_2, W_3)
    bs = (b_0, b_1, b_2, b_3)
    W_pad = jnp.zeros((4, D_IN, MAX_D), jnp.float32)
    b_pad = jnp.zeros((4, 1, MAX_D), jnp.float32)
    for t in range(4):
        W_pad = W_pad.at[t, :, :DIMS[t]].set(Ws[t])
        b_pad = b_pad.at[t, 0, :DIMS[t]].set(bs[t])

    y4 = _tc_matmul(g4, W_pad, b_pad)

    out_ref = jax.new_ref(jnp.zeros((NUM_NODES, MAX_D), jnp.float32))
    _sc_scatter(idx_0, idx_1, idx_2, idx_3, y4, out_ref)
    return out_ref[...]


# speedup vs baseline: 959.2527x; 1.2106x over previous
import functools

import jax
import jax.numpy as jnp
from jax import lax
from jax.experimental import pallas as pl
from jax.experimental.pallas import tpu as pltpu
from jax.experimental.pallas import tpu_sc as plsc

NUM_NODES = 100000
D_IN = 128
NG = 8192
DIMS = (64, 128, 32, 256)
MAX_D = 256

NC = 2
NS = 16
LANES = 16

B_ALL = 4 * NG
G_WORKERS = NC * NS
G_PER_W = B_ALL // G_WORKERS
G_CHUNK = 128
G_NCH = G_PER_W // G_CHUNK

_gather_mesh = plsc.VectorSubcoreMesh(core_axis_name="c", subcore_axis_name="s")


@functools.partial(
    pl.kernel,
    out_type=jax.ShapeDtypeStruct((B_ALL, D_IN), jnp.float32),
    mesh=_gather_mesh,
    scratch_types=[
        pltpu.VMEM((G_PER_W,), jnp.int32),
        pltpu.VMEM((2, G_CHUNK, D_IN), jnp.float32),
        pltpu.SemaphoreType.DMA,
        pltpu.SemaphoreType.DMA,
    ],
    compiler_params=pltpu.CompilerParams(use_tc_tiling_on_sc=True),
)
def _sc_gather(emb, idx, out, idxv, rows, gsem, wsem):
    wid = lax.axis_index("s") * NC + lax.axis_index("c")
    base = wid * G_PER_W
    pltpu.sync_copy(idx.at[pl.ds(base, G_PER_W)], idxv)
    wh = [None, None]
    for ch in range(G_NCH):
        b = ch % 2
        if wh[b] is not None:
            wh[b].wait()
        gh = pltpu.async_copy(
            emb.at[idxv.at[pl.ds(ch * G_CHUNK, G_CHUNK)]], rows.at[b], gsem)
        gh.wait()
        wh[b] = pltpu.async_copy(
            rows.at[b], out.at[pl.ds(base + ch * G_CHUNK, G_CHUNK)], wsem)
    wh[0].wait()
    wh[1].wait()


MM_BLK = 2048


def _mm_body(g_ref, w_ref, b_ref, y_ref):
    y_ref[...] = (
        jnp.dot(g_ref[0], w_ref[0], preferred_element_type=jnp.float32)
        + b_ref[0]
    )[None]


_tc_matmul = pl.pallas_call(
    _mm_body,
    grid=(4, NG // MM_BLK),
    in_specs=[
        pl.BlockSpec((1, MM_BLK, D_IN), lambda t, i: (t, i, 0)),
        pl.BlockSpec((1, D_IN, MAX_D), lambda t, i: (t, 0, 0)),
        pl.BlockSpec((1, 1, MAX_D), lambda t, i: (t, 0, 0)),
    ],
    out_specs=pl.BlockSpec((1, MM_BLK, MAX_D), lambda t, i: (t, i, 0)),
    out_shape=jax.ShapeDtypeStruct((4, NG, MAX_D), jnp.float32),
)


S_WORKERS = NS
S_PER_W = NG // S_WORKERS
S_CHUNK = 128
S_NCH = S_PER_W // S_CHUNK
HALF = MAX_D // 2

_scatter_mesh = plsc.VectorSubcoreMesh(core_axis_name="c", subcore_axis_name="s")


@functools.partial(
    pl.kernel,
    mesh=_scatter_mesh,
    scratch_types=[
        pltpu.VMEM((2, S_CHUNK, HALF), jnp.float32),
        pltpu.VMEM((2, S_CHUNK, HALF), jnp.float32),
        pltpu.VMEM((8, S_CHUNK), jnp.int32),
        pltpu.SemaphoreType.DMA,
        pltpu.SemaphoreType.DMA,
        pltpu.SemaphoreType.DMA,
    ],
    compiler_params=pltpu.CompilerParams(use_tc_tiling_on_sc=True),
)
def _sc_scatter(i0, i1, i2, i3, y4, out, sbuf, gbuf, idx_sc, rsem, wsem, isem):
    cid = lax.axis_index("c")
    wid = lax.axis_index("s")
    row0 = wid * S_PER_W

    def _stage_idx(idx_ref):
        hs = [pltpu.async_copy(
            idx_ref.at[pl.ds(row0 + c * S_CHUNK, S_CHUNK)], idx_sc.at[c], isem)
            for c in range(S_NCH)]
        for h in hs:
            h.wait()

    def _scatter_type(t, col0):
        cols = pl.ds(col0, HALF)
        _stage_idx((i0, i1, i2, i3)[t])

        def _read(c):
            rows = pl.ds(row0 + c * S_CHUNK, S_CHUNK)
            return pltpu.async_copy(y4.at[t, rows, cols], sbuf.at[c % 2], rsem)

        rh = [None, None]
        wh = [None, None]
        rh[0] = _read(0)
        for c in range(S_NCH):
            b = c % 2
            nb = 1 - b
            rh[b].wait()
            if c + 1 < S_NCH:
                if t != 2 and wh[nb] is not None:
                    wh[nb].wait()
                    wh[nb] = None
                rh[nb] = _read(c + 1)
            if wh[b] is not None:
                wh[b].wait()
                wh[b] = None
            if t == 2:
                pltpu.async_copy(out.at[idx_sc.at[c], cols],
                                 gbuf.at[b], rsem).wait()

                def _patch(r, carry):
                    gbuf[b, r, pl.ds(0, LANES)] = sbuf[b, r, pl.ds(0, LANES)]
                    gbuf[b, r, pl.ds(LANES, LANES)] = (
                        sbuf[b, r, pl.ds(LANES, LANES)])
                    return carry

                lax.fori_loop(0, S_CHUNK, _patch, 0)
                wh[b] = pltpu.async_copy(gbuf.at[b],
                                         out.at[idx_sc.at[c], cols], wsem)
            else:
                wh[b] = pltpu.async_copy(sbuf.at[b],
                                         out.at[idx_sc.at[c], cols], wsem)
        for h in wh:
            if h is not None:
                h.wait()

    @pl.when(cid == 0)
    def _():
        _scatter_type(0, 0)

    @pl.when(cid == 1)
    def _():
        _scatter_type(3, HALF)

    plsc.subcore_barrier()

    @pl.when(cid == 0)
    def _():
        _scatter_type(1, 0)

    plsc.subcore_barrier()

    @pl.when(cid == 0)
    def _():
        _scatter_type(2, 0)

    plsc.subcore_barrier()

    @pl.when(cid == 0)
    def _():
        _scatter_type(3, 0)


def kernel(embedding, idx_0, idx_1, idx_2, idx_3,
           W_0, b_0, W_1, b_1, W_2, b_2, W_3, b_3):
    out_ref = jax.new_ref(jnp.zeros((NUM_NODES, MAX_D), jnp.float32))

    idx_all = jnp.concatenate([idx_0, idx_1, idx_2, idx_3])
    g = _sc_gather(embedding, idx_all)
    g4 = g.reshape(4, NG, D_IN)

    Ws = (W_0, W_1, W_2, W_3)
    bs = (b_0, b_1, b_2, b_3)
    W_pad = jnp.zeros((4, D_IN, MAX_D), jnp.float32)
    b_pad = jnp.zeros((4, 1, MAX_D), jnp.float32)
    for t in range(4):
        W_pad = W_pad.at[t, :, :DIMS[t]].set(Ws[t])
        b_pad = b_pad.at[t, 0, :DIMS[t]].set(bs[t])

    y4 = _tc_matmul(g4, W_pad, b_pad)

    _sc_scatter(idx_0, idx_1, idx_2, idx_3, y4, out_ref)
    return out_ref[...]
